# baseline (speedup 1.0000x reference)
import functools

import jax
import jax.numpy as jnp
from jax import lax
from jax.experimental import pallas as pl
from jax.experimental.pallas import tpu as pltpu
from jax.experimental.pallas import tpu_sc as plsc

N_NODES = 10000
N_PAD = 10112
E_EDGES = 320000
D_IN = 128
C_OUT = 40
C_PAD = 48
DELTA = 0.5

NUM_WORKERS = 32
EDGES_PER_WORKER = E_EDGES // NUM_WORKERS
CHUNK = 125
N_CHUNKS = EDGES_PER_WORKER // CHUNK
ROWS_PER_TILE = N_PAD // 16

_MESH = plsc.VectorSubcoreMesh(core_axis_name="c", subcore_axis_name="s")
_SC_PARAMS = pltpu.CompilerParams(needs_layout_passes=False,
                                  use_tc_tiling_on_sc=False)


@functools.partial(
    pl.kernel,
    mesh=_MESH,
    out_type=jax.ShapeDtypeStruct((NUM_WORKERS, N_PAD), jnp.float32),
    compiler_params=_SC_PARAMS,
    scratch_types=[
        pltpu.VMEM((N_PAD,), jnp.float32),
        pltpu.VMEM((EDGES_PER_WORKER,), jnp.int32),
        pltpu.SemaphoreType.DMA,
    ],
)
def _deg_kernel(dst_hbm, degp_hbm, degt, idxb, sem):
    cid = lax.axis_index("c")
    sid = lax.axis_index("s")
    wid = cid * 16 + sid
    ebase = wid * EDGES_PER_WORKER

    pltpu.async_copy(dst_hbm.at[pl.ds(ebase, EDGES_PER_WORKER)], idxb, sem)

    def zero_body(i, carry):
        degt[pl.ds(i * 16, 16)] = jnp.zeros((16,), jnp.float32)
        return carry

    lax.fori_loop(0, N_PAD // 16, zero_body, 0)
    pltpu.make_async_copy(
        dst_hbm.at[pl.ds(ebase, EDGES_PER_WORKER)], idxb, sem).wait()

    ones = jnp.full((16,), 1.0, jnp.float32)

    def inner(j, carry):
        iv = idxb[pl.ds(j * 16, 16)]
        plsc.addupdate_scatter(degt, [iv], ones)
        return carry

    lax.fori_loop(0, EDGES_PER_WORKER // 16, inner, 0)
    pltpu.sync_copy(degt, degp_hbm.at[wid])


@functools.partial(
    pl.kernel,
    mesh=_MESH,
    out_type=(
        jax.ShapeDtypeStruct((N_PAD, C_PAD), jnp.float32),
        jax.ShapeDtypeStruct((N_PAD, C_PAD), jnp.float32),
    ),
    compiler_params=_SC_PARAMS,
    scratch_types=[
        pltpu.VMEM((N_CHUNKS, CHUNK), jnp.int32),
        pltpu.VMEM((N_CHUNKS, CHUNK), jnp.int32),
        [pltpu.VMEM((CHUNK, C_PAD), jnp.float32) for _ in range(10)],
        pltpu.VMEM_SHARED((N_PAD, C_PAD), jnp.float32),
        pltpu.SemaphoreType.DMA,
        pltpu.SemaphoreType.DMA,
        pltpu.SemaphoreType.DMA,
        pltpu.SemaphoreType.DMA,
    ],
)
def _scatter_kernel(g_hbm, src_hbm, dst_hbm, zeros_hbm,
                    acca_hbm, accb_hbm,
                    srcb, dstb, rows, acc_sh, gsema, gsemb, ssema, ssemb):
    cid = lax.axis_index("c")
    sid = lax.axis_index("s")
    wid = cid * 16 + sid
    r0 = sid * ROWS_PER_TILE
    gs = 5
    bufa, bufb = rows[:gs], rows[gs:]
    n_groups = N_CHUNKS // gs

    pltpu.async_copy(src_hbm.at[wid], srcb, gsema)
    pltpu.async_copy(dst_hbm.at[wid], dstb, ssema)
    pltpu.sync_copy(zeros_hbm, acc_sh.at[pl.ds(r0, ROWS_PER_TILE)])
    pltpu.make_async_copy(src_hbm.at[wid], srcb, gsema).wait()
    pltpu.make_async_copy(dst_hbm.at[wid], dstb, ssema).wait()
    plsc.subcore_barrier()

    def fire_gathers(g, bufs, sem):
        return [pltpu.async_copy(g_hbm.at[srcb.at[g * gs + b]], bufs[b], sem)
                for b in range(gs)]

    def do_scatters(g, bufs, sem):
        sd = [pltpu.async_copy(bufs[b], acc_sh.at[dstb.at[g * gs + b]],
                               sem, add=True)
              for b in range(gs)]
        for d in sd:
            d.wait()

    def drain_gathers(bufs, sem):
        for b in range(gs):
            pltpu.make_async_copy(g_hbm.at[srcb.at[0]], bufs[b], sem).wait()

    fire_gathers(0, bufa, gsema)

    def pair(j, carry):
        g0 = 2 * j
        fire_gathers(g0 + 1, bufb, gsemb)
        drain_gathers(bufa, gsema)
        do_scatters(g0, bufa, ssema)

        @pl.when(j + 1 < n_groups // 2)
        def _():
            fire_gathers(g0 + 2, bufa, gsema)

        drain_gathers(bufb, gsemb)
        do_scatters(g0 + 1, bufb, ssemb)
        return carry

    lax.fori_loop(0, n_groups // 2, pair, 0)
    plsc.subcore_barrier()

    @pl.when(cid == 0)
    def _():
        pltpu.sync_copy(acc_sh.at[pl.ds(r0, ROWS_PER_TILE)],
                        acca_hbm.at[pl.ds(r0, ROWS_PER_TILE)])

    @pl.when(cid == 1)
    def _():
        pltpu.sync_copy(acc_sh.at[pl.ds(r0, ROWS_PER_TILE)],
                        accb_hbm.at[pl.ds(r0, ROWS_PER_TILE)])


def _tcz_body(featp_ref, wt_ref, z_ref):
    z_ref[...] = jnp.dot(featp_ref[...], wt_ref[...],
                         preferred_element_type=jnp.float32)


def _tcn_body(z_ref, degp_ref, g0_ref, norm_ref):
    deg = lax.dot_general(
        degp_ref[...], jnp.ones((NUM_WORKERS, 1), jnp.float32),
        (((0,), (0,)), ((), ())),
        preferred_element_type=jnp.float32) + 1.0
    nrm = lax.rsqrt(deg)
    g0_ref[...] = z_ref[...] * nrm
    norm_ref[...] = nrm


def _tc2_body(h_ref, norm_ref, a_ref, b_ref, h1_ref, g1_ref):
    n = norm_ref[...]
    h = h_ref[...]
    s = a_ref[...] + b_ref[...]
    h1 = (1.0 - DELTA) * h + DELTA * (n * n * h + n * s)
    h1_ref[...] = h1
    g1_ref[...] = n * h1


def _tc3_body(h_ref, norm_ref, a_ref, b_ref, bias_ref, out_ref):
    n = norm_ref[...]
    h = h_ref[...]
    s = a_ref[...] + b_ref[...]
    out = (1.0 - DELTA) * h + DELTA * (n * n * h + n * s)
    out_ref[...] = out[:N_NODES, :C_OUT] + bias_ref[...]


_tcz = pl.pallas_call(
    _tcz_body,
    out_shape=jax.ShapeDtypeStruct((N_PAD, C_PAD), jnp.float32),
)

_tcn = pl.pallas_call(
    _tcn_body,
    out_shape=(
        jax.ShapeDtypeStruct((N_PAD, C_PAD), jnp.float32),
        jax.ShapeDtypeStruct((N_PAD, 1), jnp.float32),
    ),
)

_tc2 = pl.pallas_call(
    _tc2_body,
    out_shape=(
        jax.ShapeDtypeStruct((N_PAD, C_PAD), jnp.float32),
        jax.ShapeDtypeStruct((N_PAD, C_PAD), jnp.float32),
    ),
)

_tc3 = pl.pallas_call(
    _tc3_body,
    out_shape=jax.ShapeDtypeStruct((N_NODES, C_OUT), jnp.float32),
)


def kernel(feat, edge_index, W, b):
    src = edge_index[0]
    dst = edge_index[1]
    src3 = src.reshape(NUM_WORKERS, N_CHUNKS, CHUNK)
    dst3 = dst.reshape(NUM_WORKERS, N_CHUNKS, CHUNK)

    wt = jnp.pad(W, ((0, C_PAD - C_OUT), (0, 0))).T
    bias = b[None, :]
    zeros = jnp.zeros((ROWS_PER_TILE, C_PAD), jnp.float32)
    featp = jnp.pad(feat, ((0, N_PAD - N_NODES), (0, 0)))

    degp = _deg_kernel(dst)
    z = _tcz(featp, wt)
    g0, norm = _tcn(z, degp)
    a1, b1 = _scatter_kernel(g0, src3, dst3, zeros)
    h1, g1 = _tc2(z, norm, a1, b1)
    a2, b2 = _scatter_kernel(g1, src3, dst3, zeros)
    return _tc3(h1, norm, a2, b2, bias)


# speedup vs baseline: 1.0246x; 1.0246x over previous
import functools

import jax
import jax.numpy as jnp
from jax import lax
from jax.experimental import pallas as pl
from jax.experimental.pallas import tpu as pltpu
from jax.experimental.pallas import tpu_sc as plsc

N_NODES = 10000
N_PAD = 10112
E_EDGES = 320000
D_IN = 128
C_OUT = 40
C_PAD = 48
DELTA = 0.5

NUM_WORKERS = 32
EDGES_PER_WORKER = 10240
E_TOTAL = NUM_WORKERS * EDGES_PER_WORKER
CHUNK = 128
N_CHUNKS = EDGES_PER_WORKER // CHUNK
ROWS_PER_TILE = N_PAD // 16

_MESH = plsc.VectorSubcoreMesh(core_axis_name="c", subcore_axis_name="s")
_SC_PARAMS = pltpu.CompilerParams(needs_layout_passes=False,
                                  use_tc_tiling_on_sc=False)


@functools.partial(
    pl.kernel,
    mesh=_MESH,
    out_type=jax.ShapeDtypeStruct((NUM_WORKERS, N_PAD), jnp.float32),
    compiler_params=_SC_PARAMS,
    scratch_types=[
        pltpu.VMEM((N_PAD,), jnp.float32),
        pltpu.VMEM((EDGES_PER_WORKER,), jnp.int32),
        pltpu.SemaphoreType.DMA,
    ],
)
def _deg_kernel(dst_hbm, degp_hbm, degt, idxb, sem):
    cid = lax.axis_index("c")
    sid = lax.axis_index("s")
    wid = cid * 16 + sid
    ebase = wid * EDGES_PER_WORKER

    pltpu.async_copy(dst_hbm.at[pl.ds(ebase, EDGES_PER_WORKER)], idxb, sem)

    def zero_body(i, carry):
        degt[pl.ds(i * 16, 16)] = jnp.zeros((16,), jnp.float32)
        return carry

    lax.fori_loop(0, N_PAD // 16, zero_body, 0)
    pltpu.make_async_copy(
        dst_hbm.at[pl.ds(ebase, EDGES_PER_WORKER)], idxb, sem).wait()

    ones = jnp.full((16,), 1.0, jnp.float32)

    def inner(j, carry):
        iv = idxb[pl.ds(j * 16, 16)]
        plsc.addupdate_scatter(degt, [iv], ones)
        return carry

    lax.fori_loop(0, EDGES_PER_WORKER // 16, inner, 0)
    pltpu.sync_copy(degt, degp_hbm.at[wid])


@functools.partial(
    pl.kernel,
    mesh=_MESH,
    out_type=(
        jax.ShapeDtypeStruct((N_PAD, C_PAD), jnp.float32),
        jax.ShapeDtypeStruct((N_PAD, C_PAD), jnp.float32),
    ),
    compiler_params=_SC_PARAMS,
    scratch_types=[
        pltpu.VMEM((N_CHUNKS, CHUNK), jnp.int32),
        pltpu.VMEM((N_CHUNKS, CHUNK), jnp.int32),
        [pltpu.VMEM((CHUNK, C_PAD), jnp.float32) for _ in range(10)],
        pltpu.VMEM_SHARED((N_PAD, C_PAD), jnp.float32),
        pltpu.SemaphoreType.DMA,
        pltpu.SemaphoreType.DMA,
        pltpu.SemaphoreType.DMA,
        pltpu.SemaphoreType.DMA,
    ],
)
def _scatter_kernel(g_hbm, src_hbm, dst_hbm, zeros_hbm,
                    acca_hbm, accb_hbm,
                    srcb, dstb, rows, acc_sh, gsema, gsemb, ssema, ssemb):
    cid = lax.axis_index("c")
    sid = lax.axis_index("s")
    wid = cid * 16 + sid
    r0 = sid * ROWS_PER_TILE
    gs = 5
    bufa, bufb = rows[:gs], rows[gs:]
    n_groups = N_CHUNKS // gs

    pltpu.async_copy(src_hbm.at[wid], srcb, gsema)
    pltpu.async_copy(dst_hbm.at[wid], dstb, ssema)
    pltpu.sync_copy(zeros_hbm, acc_sh.at[pl.ds(r0, ROWS_PER_TILE)])
    pltpu.make_async_copy(src_hbm.at[wid], srcb, gsema).wait()
    pltpu.make_async_copy(dst_hbm.at[wid], dstb, ssema).wait()
    plsc.subcore_barrier()

    def fire_gathers(g, bufs, sem):
        return [pltpu.async_copy(g_hbm.at[srcb.at[g * gs + b]], bufs[b], sem)
                for b in range(gs)]

    def do_scatters(g, bufs, sem):
        sd = [pltpu.async_copy(bufs[b], acc_sh.at[dstb.at[g * gs + b]],
                               sem, add=True)
              for b in range(gs)]
        for d in sd:
            d.wait()

    def drain_gathers(bufs, sem):
        for b in range(gs):
            pltpu.make_async_copy(g_hbm.at[srcb.at[0]], bufs[b], sem).wait()

    fire_gathers(0, bufa, gsema)

    def pair(j, carry):
        g0 = 2 * j
        fire_gathers(g0 + 1, bufb, gsemb)
        drain_gathers(bufa, gsema)
        do_scatters(g0, bufa, ssema)

        @pl.when(j + 1 < n_groups // 2)
        def _():
            fire_gathers(g0 + 2, bufa, gsema)

        drain_gathers(bufb, gsemb)
        do_scatters(g0 + 1, bufb, ssemb)
        return carry

    lax.fori_loop(0, n_groups // 2, pair, 0)
    plsc.subcore_barrier()

    @pl.when(cid == 0)
    def _():
        pltpu.sync_copy(acc_sh.at[pl.ds(r0, ROWS_PER_TILE)],
                        acca_hbm.at[pl.ds(r0, ROWS_PER_TILE)])

    @pl.when(cid == 1)
    def _():
        pltpu.sync_copy(acc_sh.at[pl.ds(r0, ROWS_PER_TILE)],
                        accb_hbm.at[pl.ds(r0, ROWS_PER_TILE)])


def _tcz_body(featp_ref, wt_ref, z_ref):
    z_ref[...] = jnp.dot(featp_ref[...], wt_ref[...],
                         preferred_element_type=jnp.float32)


def _tcn_body(z_ref, degpt_ref, g0_ref, norm_ref):
    deg = jnp.sum(degpt_ref[...], axis=1, keepdims=True) + 1.0
    nrm = lax.rsqrt(deg)
    g0_ref[...] = z_ref[...] * nrm
    norm_ref[...] = nrm


def _tc2_body(h_ref, norm_ref, a_ref, b_ref, h1_ref, g1_ref):
    n = norm_ref[...]
    h = h_ref[...]
    s = a_ref[...] + b_ref[...]
    h1 = (1.0 - DELTA) * h + DELTA * (n * n * h + n * s)
    h1_ref[...] = h1
    g1_ref[...] = n * h1


def _tc3_body(h_ref, norm_ref, a_ref, b_ref, bias_ref, out_ref):
    n = norm_ref[...]
    h = h_ref[...]
    s = a_ref[...] + b_ref[...]
    out = (1.0 - DELTA) * h + DELTA * (n * n * h + n * s)
    out_ref[...] = out[:N_NODES, :C_OUT] + bias_ref[...]


_tcz = pl.pallas_call(
    _tcz_body,
    out_shape=jax.ShapeDtypeStruct((N_PAD, C_PAD), jnp.float32),
)

_tcn = pl.pallas_call(
    _tcn_body,
    out_shape=(
        jax.ShapeDtypeStruct((N_PAD, C_PAD), jnp.float32),
        jax.ShapeDtypeStruct((N_PAD, 1), jnp.float32),
    ),
)

_tc2 = pl.pallas_call(
    _tc2_body,
    out_shape=(
        jax.ShapeDtypeStruct((N_PAD, C_PAD), jnp.float32),
        jax.ShapeDtypeStruct((N_PAD, C_PAD), jnp.float32),
    ),
)

_tc3 = pl.pallas_call(
    _tc3_body,
    out_shape=jax.ShapeDtypeStruct((N_NODES, C_OUT), jnp.float32),
)


def kernel(feat, edge_index, W, b):
    src = edge_index[0]
    dst = edge_index[1]
    pad = N_NODES + jnp.arange(E_TOTAL - E_EDGES, dtype=jnp.int32) % (
        N_PAD - N_NODES)
    srcp = jnp.concatenate([src, pad])
    dstp = jnp.concatenate([dst, pad])
    src3 = srcp.reshape(NUM_WORKERS, N_CHUNKS, CHUNK)
    dst3 = dstp.reshape(NUM_WORKERS, N_CHUNKS, CHUNK)

    wt = jnp.pad(W, ((0, C_PAD - C_OUT), (0, 0))).T
    bias = b[None, :]
    zeros = jnp.zeros((ROWS_PER_TILE, C_PAD), jnp.float32)
    featp = jnp.pad(feat, ((0, N_PAD - N_NODES), (0, 0)))

    degp = _deg_kernel(dstp)
    z = _tcz(featp, wt)
    g0, norm = _tcn(z, degp.T)
    a1, b1 = _scatter_kernel(g0, src3, dst3, zeros)
    h1, g1 = _tc2(z, norm, a1, b1)
    a2, b2 = _scatter_kernel(g1, src3, dst3, zeros)
    return _tc3(h1, norm, a2, b2, bias)
